# Initial kernel scaffold; baseline (speedup 1.0000x reference)
import jax
import jax.numpy as jnp
from jax.experimental import pallas as pl


def kernel(hidden_states, weight):
    raise NotImplementedError("write your pallas kernel here")


# speedup vs baseline: 3.1166x; 3.1166x over previous
import functools

import jax
import jax.numpy as jnp
from jax import lax
from jax.experimental import pallas as pl
from jax.experimental.pallas import tpu as pltpu
from jax.experimental.pallas import tpu_sc as plsc

N_EXPERTS = 160
TOP_K = 6


_BT = 512


def _matmul_body(w_ref, x_ref, out_ref):
    out_ref[...] = lax.dot_general(
        w_ref[...], x_ref[...],
        dimension_numbers=(((1,), (1,)), ((), ())),
        preferred_element_type=jnp.float32,
    )


def _logits_t(x, weight):
    n_tok, h = x.shape
    grid = n_tok // _BT
    return pl.pallas_call(
        _matmul_body,
        grid=(grid,),
        in_specs=[
            pl.BlockSpec((N_EXPERTS, h), lambda i: (0, 0)),
            pl.BlockSpec((_BT, h), lambda i: (i, 0)),
        ],
        out_specs=pl.BlockSpec((N_EXPERTS, _BT), lambda i: (0, i)),
        out_shape=jax.ShapeDtypeStruct((N_EXPERTS, n_tok), jnp.float32),
    )(weight, x)


def _insert(tv, ti, v, i):
    nv, ni = [], []
    cv, ci = v, i
    for j in range(TOP_K):
        c = cv > tv[j]
        nv.append(jnp.where(c, cv, tv[j]))
        ni.append(jnp.where(c, ci, ti[j]))
        cv = jnp.where(c, tv[j], cv)
        ci = jnp.where(c, ti[j], ci)
    return tuple(nv), tuple(ni)


_HALF = 512
_UNROLL = 4


def _sc_topk(logits_t):
    n_exp, n_tok = logits_t.shape
    info = plsc.get_sparse_core_info()
    nc, ns = info.num_cores, info.num_subcores
    nw = nc * ns
    rows_per_w = n_tok // nw
    n_half = rows_per_w // _HALF
    n_groups = _HALF // 16

    mesh = plsc.VectorSubcoreMesh(core_axis_name="c", subcore_axis_name="s")

    @functools.partial(
        pl.kernel,
        mesh=mesh,
        out_type=[
            jax.ShapeDtypeStruct((n_tok * TOP_K,), jnp.int32),
            jax.ShapeDtypeStruct((n_tok * TOP_K,), jnp.float32),
        ],
        scratch_types=[
            pltpu.VMEM((n_exp, _HALF), jnp.float32),
            pltpu.VMEM((rows_per_w * TOP_K,), jnp.int32),
            pltpu.VMEM((rows_per_w * TOP_K,), jnp.float32),
        ],
    )
    def k(logits_hbm, oidx_hbm, ow_hbm, buf, oi, ow):
        wid = lax.axis_index("s") * nc + lax.axis_index("c")

        for half in range(n_half):
            col0 = pl.multiple_of(wid * rows_per_w + half * _HALF, _HALF)
            pltpu.sync_copy(logits_hbm.at[:, pl.ds(col0, _HALF)], buf)

            def group_body(g, _):
                base = g * 16
                tv0 = tuple(jnp.full((16,), -jnp.inf, jnp.float32) for _ in range(TOP_K))
                ti0 = tuple(jnp.zeros((16,), jnp.int32) for _ in range(TOP_K))

                def exp_body(i, t):
                    tv, ti = t
                    for u in range(_UNROLL):
                        e = i * _UNROLL + u
                        v = buf[e, pl.ds(base, 16)]
                        ev = jnp.broadcast_to(e, (16,)).astype(jnp.int32)
                        tv, ti = _insert(tv, ti, v, ev)
                    return tv, ti

                vals, eidx = lax.fori_loop(0, n_exp // _UNROLL, exp_body, (tv0, ti0))

                exps = [jnp.exp(vj - vals[0]) for vj in vals]
                s = exps[0]
                for j in range(1, TOP_K):
                    s = s + exps[j]
                grp = half * n_groups + g
                off = grp * (TOP_K * 16)
                for j in range(TOP_K):
                    oi[pl.ds(off + j * 16, 16)] = eidx[j]
                    ow[pl.ds(off + j * 16, 16)] = exps[j] / s
                return 0

            lax.fori_loop(0, n_groups, group_body, 0)

        out0 = wid * rows_per_w * TOP_K
        pltpu.sync_copy(oi, oidx_hbm.at[pl.ds(out0, rows_per_w * TOP_K)])
        pltpu.sync_copy(ow, ow_hbm.at[pl.ds(out0, rows_per_w * TOP_K)])

    return k(logits_t)


def kernel(hidden_states, weight):
    b, s, h = hidden_states.shape
    x = hidden_states.reshape(b * s, h)
    logits_t = _logits_t(x, weight)
    idx_flat, w_flat = _sc_topk(logits_t)
    n_tok = b * s
    def _assemble(a):
        return a.reshape(n_tok // 16, TOP_K, 16).transpose(0, 2, 1).reshape(n_tok, TOP_K)
    return _assemble(idx_flat), _assemble(w_flat)
